# baseline (speedup 1.0000x reference)
import functools

import jax
import jax.numpy as jnp
from jax import lax
from jax.experimental import pallas as pl
from jax.experimental.pallas import tpu as pltpu
from jax.experimental.pallas import tpu_sc as plsc

_NUM_CLASSES = 1000
_BATCH = 16384
_NC = 2
_NS = 16
_NW = _NC * _NS
_ROWS_PER_W = _BATCH // _NW
_CHUNK = 128
_N_CHUNKS = _ROWS_PER_W // _CHUNK
_GROUPS = _CHUNK // 16
_BUF_WORDS = _CHUNK * _NUM_CLASSES


def _sc_body(label_hbm, out_hbm, label_v, buf_v):
    wid = lax.axis_index("s") * _NC + lax.axis_index("c")
    row0 = wid * _ROWS_PER_W
    pltpu.sync_copy(label_hbm.at[pl.ds(row0, _ROWS_PER_W)], label_v)

    zeros = jnp.zeros((16,), jnp.int32)
    ones = jnp.ones((16,), jnp.int32)
    iota = lax.iota(jnp.int32, 16)

    def zero_body(i, carry):
        buf_v[pl.ds(i * 16, 16)] = zeros
        return carry

    lax.fori_loop(0, _BUF_WORDS // 16, zero_body, 0)

    for c in range(_N_CHUNKS):
        idxs = []
        for g in range(_GROUPS):
            lv = label_v[pl.ds(c * _CHUNK + g * 16, 16)]
            idx = (g * 16 + iota) * _NUM_CLASSES + lv
            idxs.append(idx)
            plsc.store_scatter(buf_v, [idx], ones)
        dst0 = (row0 + c * _CHUNK) * _NUM_CLASSES
        pltpu.sync_copy(buf_v, out_hbm.at[pl.ds(dst0, _BUF_WORDS)])
        if c < _N_CHUNKS - 1:
            for idx in idxs:
                plsc.store_scatter(buf_v, [idx], zeros)


_one_hot_sc = functools.partial(
    pl.kernel,
    out_type=jax.ShapeDtypeStruct((_BATCH * _NUM_CLASSES,), jnp.int32),
    mesh=plsc.VectorSubcoreMesh(core_axis_name="c", subcore_axis_name="s"),
    compiler_params=pltpu.CompilerParams(needs_layout_passes=False),
    scratch_types=[
        pltpu.VMEM((_ROWS_PER_W,), jnp.int32),
        pltpu.VMEM((_BUF_WORDS,), jnp.int32),
    ],
)(_sc_body)


def kernel(label):
    flat = _one_hot_sc(label)
    return flat.reshape(_BATCH, _NUM_CLASSES)


# speedup vs baseline: 1.1964x; 1.1964x over previous
import functools

import jax
import jax.numpy as jnp
from jax import lax
from jax.experimental import pallas as pl
from jax.experimental.pallas import tpu as pltpu
from jax.experimental.pallas import tpu_sc as plsc

_NUM_CLASSES = 1000
_BATCH = 16384
_NC = 2
_NS = 16
_NW = _NC * _NS
_ROWS_PER_W = _BATCH // _NW
_CHUNK = 128
_N_CHUNKS = _ROWS_PER_W // _CHUNK
_GROUPS = _CHUNK // 16
_BUF_WORDS = _CHUNK * _NUM_CLASSES


_ZUNROLL = 16


def _sc_body(label_hbm, out_hbm, label_v, buf_v, lsem):
    wid = lax.axis_index("s") * _NC + lax.axis_index("c")
    row0 = wid * _ROWS_PER_W
    lcopy = pltpu.make_async_copy(
        label_hbm.at[pl.ds(row0, _ROWS_PER_W)], label_v, lsem
    )
    lcopy.start()

    zeros = jnp.zeros((16,), jnp.int32)
    ones = jnp.ones((16,), jnp.int32)
    iota = lax.iota(jnp.int32, 16)

    def zero_body(i, carry):
        for u in range(_ZUNROLL):
            buf_v[pl.ds((i * _ZUNROLL + u) * 16, 16)] = zeros
        return carry

    lax.fori_loop(0, _BUF_WORDS // (16 * _ZUNROLL), zero_body, 0)
    lcopy.wait()

    for c in range(_N_CHUNKS):
        idxs = []
        for g in range(_GROUPS):
            lv = label_v[pl.ds(c * _CHUNK + g * 16, 16)]
            idx = (g * 16 + iota) * _NUM_CLASSES + lv
            idxs.append(idx)
            plsc.store_scatter(buf_v, [idx], ones)
        dst0 = (row0 + c * _CHUNK) * _NUM_CLASSES
        pltpu.sync_copy(buf_v, out_hbm.at[pl.ds(dst0, _BUF_WORDS)])
        if c < _N_CHUNKS - 1:
            for idx in idxs:
                plsc.store_scatter(buf_v, [idx], zeros)


_one_hot_sc = functools.partial(
    pl.kernel,
    out_type=jax.ShapeDtypeStruct((_BATCH * _NUM_CLASSES,), jnp.int32),
    mesh=plsc.VectorSubcoreMesh(core_axis_name="c", subcore_axis_name="s"),
    compiler_params=pltpu.CompilerParams(needs_layout_passes=False),
    scratch_types=[
        pltpu.VMEM((_ROWS_PER_W,), jnp.int32),
        pltpu.VMEM((_BUF_WORDS,), jnp.int32),
        pltpu.SemaphoreType.DMA,
    ],
)(_sc_body)


def kernel(label):
    flat = _one_hot_sc(label)
    return flat.reshape(_BATCH, _NUM_CLASSES)


# speedup vs baseline: 1.9501x; 1.6299x over previous
import functools

import jax
import jax.numpy as jnp
from jax import lax
from jax.experimental import pallas as pl
from jax.experimental.pallas import tpu as pltpu
from jax.experimental.pallas import tpu_sc as plsc

_NUM_CLASSES = 1000
_BATCH = 16384
_NC = 2
_NS = 16
_NW = _NC * _NS
_ROWS_PER_W = _BATCH // _NW
_CHUNK = 64
_N_CHUNKS = _ROWS_PER_W // _CHUNK
_GROUPS = _CHUNK // 16
_FULL = _NUM_CLASSES // 16
_TAIL = _NUM_CLASSES - _FULL * 16


def _sc_body(label_hbm, out_hbm, label_v, buf_v, lsem):
    wid = lax.axis_index("s") * _NC + lax.axis_index("c")
    row0 = wid * _ROWS_PER_W
    lcopy = pltpu.make_async_copy(
        label_hbm.at[pl.ds(row0, _ROWS_PER_W)], label_v, lsem
    )
    lcopy.start()

    zeros = jnp.zeros((16,), jnp.int32)
    ones = jnp.ones((16,), jnp.int32)
    iota = lax.iota(jnp.int32, 16)
    tail_col = _FULL * 16 + iota
    tail_mask = iota < _TAIL

    def zero_body(r, carry):
        for g in range(_FULL):
            buf_v[r, pl.ds(g * 16, 16)] = zeros
        row_splat = jnp.full((16,), 0, jnp.int32) + r
        plsc.store_scatter(
            buf_v, [row_splat, tail_col], zeros, mask=tail_mask
        )
        return carry

    lax.fori_loop(0, _CHUNK, zero_body, 0)
    lcopy.wait()

    for c in range(_N_CHUNKS):
        idxs = []
        for g in range(_GROUPS):
            lv = label_v[pl.ds(c * _CHUNK + g * 16, 16)]
            row16 = g * 16 + iota
            idxs.append((row16, lv))
            plsc.store_scatter(buf_v, [row16, lv], ones)
        pltpu.sync_copy(buf_v, out_hbm.at[pl.ds(row0 + c * _CHUNK, _CHUNK)])
        if c < _N_CHUNKS - 1:
            for row16, lv in idxs:
                plsc.store_scatter(buf_v, [row16, lv], zeros)


_one_hot_sc = functools.partial(
    pl.kernel,
    out_type=jax.ShapeDtypeStruct((_BATCH, _NUM_CLASSES), jnp.int32),
    mesh=plsc.VectorSubcoreMesh(core_axis_name="c", subcore_axis_name="s"),
    compiler_params=pltpu.CompilerParams(needs_layout_passes=False),
    scratch_types=[
        pltpu.VMEM((_ROWS_PER_W,), jnp.int32),
        pltpu.VMEM((_CHUNK, _NUM_CLASSES), jnp.int32),
        pltpu.SemaphoreType.DMA,
    ],
)(_sc_body)


def kernel(label):
    return _one_hot_sc(label)
